# baseline (speedup 1.0000x reference)
import functools

import jax
import jax.numpy as jnp
from jax import lax
from jax.experimental import pallas as pl
from jax.experimental.pallas import tpu as pltpu
from jax.experimental.pallas import tpu_sc as plsc

L = 16
NC = 2
NS = 16
NW = NC * NS

TOTAL = 32768
F = 128
B_PER_W = TOTAL // NW
CHUNK = 128
N_CHUNKS = B_PER_W // CHUNK


def _sc_body(hits_hbm, cond_hbm, gid_hbm, out_hbm,
             idx_v, hits_v, cond_v, out_v, hsem, gsem, osem):
    wid = lax.axis_index("s") * NC + lax.axis_index("c")
    base = wid * B_PER_W

    pltpu.sync_copy(gid_hbm.at[pl.ds(wid * N_CHUNKS, N_CHUNKS)], idx_v)

    def start_inputs(j):
        b = j & 1
        row0 = base + j * CHUNK
        pltpu.async_copy(cond_hbm.at[idx_v.at[j]], cond_v.at[b], gsem)
        pltpu.async_copy(hits_hbm.at[pl.ds(row0, CHUNK)], hits_v.at[b], hsem)

    start_inputs(0)
    start_inputs(1)

    stores = []
    for j in range(N_CHUNKS):
        b = j & 1
        row0 = base + j * CHUNK
        pltpu.make_async_copy(cond_hbm.at[idx_v.at[j]], cond_v.at[b], gsem).wait()
        pltpu.make_async_copy(hits_hbm.at[pl.ds(row0, CHUNK)], hits_v.at[b],
                              hsem).wait()
        if j >= 2:
            stores[j - 2].wait()

        @plsc.parallel_loop(0, CHUNK, 1, unroll=2)
        def sub_row(r):
            for g in range(F // L):
                s = pl.ds(g * L, L)
                out_v[b, r, s] = hits_v[b, r, s] - cond_v[b, r, s]
        stores.append(
            pltpu.async_copy(out_v.at[b], out_hbm.at[pl.ds(row0, CHUNK)], osem))
        if j + 2 < N_CHUNKS:
            start_inputs(j + 2)

    stores[N_CHUNKS - 2].wait()
    stores[N_CHUNKS - 1].wait()


@jax.jit
def _call(hits, cond, gid2d):
    mesh = plsc.VectorSubcoreMesh(core_axis_name="c", subcore_axis_name="s")
    k = pl.kernel(
        _sc_body,
        mesh=mesh,
        out_type=jax.ShapeDtypeStruct((TOTAL, F), jnp.float32),
        scratch_types=[
            pltpu.VMEM((N_CHUNKS, CHUNK), jnp.int32),
            pltpu.VMEM((2, CHUNK, F), jnp.float32),
            pltpu.VMEM((2, CHUNK, F), jnp.float32),
            pltpu.VMEM((2, CHUNK, F), jnp.float32),
            pltpu.SemaphoreType.DMA,
            pltpu.SemaphoreType.DMA,
            pltpu.SemaphoreType.DMA,
        ],
    )
    return k(hits, cond, gid2d)


def kernel(hits, cond, group_ids):
    gid2d = group_ids.astype(jnp.int32).reshape(TOTAL // CHUNK, CHUNK)
    return _call(hits, cond, gid2d)


# speedup vs baseline: 1.3175x; 1.3175x over previous
import functools

import jax
import jax.numpy as jnp
from jax import lax
from jax.experimental import pallas as pl
from jax.experimental.pallas import tpu as pltpu
from jax.experimental.pallas import tpu_sc as plsc

L = 16
NC = 2
NS = 16
NW = NC * NS

TOTAL = 32768
F = 128
N_GROUPS = 2048
B_PER_W = TOTAL // NW
CHUNK = 128
N_CHUNKS = B_PER_W // CHUNK
CAP = 384


def _sc_body(hits_hbm, cond_hbm, gid_hbm, out_hbm,
             idx_v, cond_v, hits_v, out_v, csem, hsem, gsem, osem):
    wid = lax.axis_index("s") * NC + lax.axis_index("c")
    base = wid * B_PER_W

    pltpu.sync_copy(gid_hbm.at[pl.ds(wid * N_CHUNKS, N_CHUNKS)], idx_v)

    lo = idx_v[0, pl.ds(0, L)][0]
    hi = idx_v[N_CHUNKS - 1, pl.ds(CHUNK - L, L)][L - 1]
    start = pl.multiple_of(jnp.minimum(lo, N_GROUPS - CAP) & ~7, 8)

    def start_hits(j):
        pltpu.async_copy(hits_hbm.at[pl.ds(base + j * CHUNK, CHUNK)],
                         hits_v.at[j & 1], hsem)

    @pl.when(hi - start < CAP)
    def _fast():
        cload = pltpu.async_copy(cond_hbm.at[pl.ds(start, CAP)], cond_v, csem)
        start_hits(0)
        start_hits(1)
        cload.wait()

        iota = lax.iota(jnp.int32, L)
        cols = [iota + (g * L) for g in range(F // L)]

        stores = []
        for j in range(N_CHUNKS):
            b = j & 1
            row0 = base + j * CHUNK
            pltpu.make_async_copy(hits_hbm.at[pl.ds(row0, CHUNK)],
                                  hits_v.at[b], hsem).wait()
            if j >= 2:
                stores[j - 2].wait()

            @plsc.parallel_loop(0, CHUNK, 1)
            def row_body(r):
                gidm = idx_v[j, pl.ds(r & ~(L - 1), L)] - start
                lane = jnp.broadcast_to(r & (L - 1), (L,))
                rowv = jnp.take_along_axis(gidm, lane, axis=0)
                for g in range(F // L):
                    s = pl.ds(g * L, L)
                    cval = plsc.load_gather(cond_v, [rowv, cols[g]])
                    out_v[b, r, s] = hits_v[b, r, s] - cval

            stores.append(pltpu.async_copy(
                out_v.at[b], out_hbm.at[pl.ds(row0, CHUNK)], osem))
            if j + 2 < N_CHUNKS:
                start_hits(j + 2)

        stores[N_CHUNKS - 2].wait()
        stores[N_CHUNKS - 1].wait()

    @pl.when(hi - start >= CAP)
    def _general():
        for j in range(N_CHUNKS):
            row0 = base + j * CHUNK
            gather = pltpu.async_copy(cond_hbm.at[idx_v.at[j]],
                                      cond_v.at[pl.ds(0, CHUNK)], gsem)
            pltpu.sync_copy(hits_hbm.at[pl.ds(row0, CHUNK)], hits_v.at[0])
            gather.wait()

            @plsc.parallel_loop(0, CHUNK, 1)
            def sub_row(r):
                for g in range(F // L):
                    s = pl.ds(g * L, L)
                    out_v[0, r, s] = hits_v[0, r, s] - cond_v[r, s]

            pltpu.sync_copy(out_v.at[0], out_hbm.at[pl.ds(row0, CHUNK)])


@jax.jit
def _call(hits, cond, gid2d):
    mesh = plsc.VectorSubcoreMesh(core_axis_name="c", subcore_axis_name="s")
    k = pl.kernel(
        _sc_body,
        mesh=mesh,
        compiler_params=pltpu.CompilerParams(needs_layout_passes=False),
        out_type=jax.ShapeDtypeStruct((TOTAL, F), jnp.float32),
        scratch_types=[
            pltpu.VMEM((N_CHUNKS, CHUNK), jnp.int32),
            pltpu.VMEM((CAP, F), jnp.float32),
            pltpu.VMEM((2, CHUNK, F), jnp.float32),
            pltpu.VMEM((2, CHUNK, F), jnp.float32),
            pltpu.SemaphoreType.DMA,
            pltpu.SemaphoreType.DMA,
            pltpu.SemaphoreType.DMA,
            pltpu.SemaphoreType.DMA,
        ],
    )
    return k(hits, cond, gid2d)


def kernel(hits, cond, group_ids):
    gid2d = group_ids.astype(jnp.int32).reshape(TOTAL // CHUNK, CHUNK)
    return _call(hits, cond, gid2d)


# speedup vs baseline: 1.4722x; 1.1174x over previous
import functools

import jax
import jax.numpy as jnp
from jax import lax
from jax.experimental import pallas as pl
from jax.experimental.pallas import tpu as pltpu
from jax.experimental.pallas import tpu_sc as plsc

L = 16
NC = 2
NS = 16
NW = NC * NS

TOTAL = 32768
F = 128
N_GROUPS = 2048
B_PER_W = TOTAL // NW
CHUNK = 128
N_CHUNKS = B_PER_W // CHUNK
CAP = 128
NBUF = 3


def _sc_body(hits_hbm, cond_hbm, gid_hbm, out_hbm,
             idx_v, cond_v, hits_v, out_v, csem, hsem, gsem, osem):
    wid = lax.axis_index("s") * NC + lax.axis_index("c")
    base = wid * B_PER_W

    pltpu.sync_copy(gid_hbm.at[pl.ds(wid * N_CHUNKS, N_CHUNKS)], idx_v)

    lo = idx_v[0, pl.ds(0, L)][0]
    hi = idx_v[N_CHUNKS - 1, pl.ds(CHUNK - L, L)][L - 1]
    start = pl.multiple_of(jnp.minimum(lo, N_GROUPS - CAP) & ~7, 8)

    def start_hits(j):
        pltpu.async_copy(hits_hbm.at[pl.ds(base + j * CHUNK, CHUNK)],
                         hits_v.at[j % NBUF], hsem)

    @pl.when(hi - start < CAP)
    def _fast():
        cload = pltpu.async_copy(cond_hbm.at[pl.ds(start, CAP)], cond_v, csem)
        for j in range(NBUF):
            start_hits(j)
        cload.wait()

        iota = lax.iota(jnp.int32, L)
        cols = [iota + (g * L) for g in range(F // L)]

        stores = []
        for j in range(N_CHUNKS):
            b = j % NBUF
            row0 = base + j * CHUNK
            pltpu.make_async_copy(hits_hbm.at[pl.ds(row0, CHUNK)],
                                  hits_v.at[b], hsem).wait()
            if j >= NBUF:
                stores[j - NBUF].wait()

            @plsc.parallel_loop(0, CHUNK, 1)
            def row_body(r):
                gidm = idx_v[j, pl.ds(r & ~(L - 1), L)] - start
                lane = jnp.broadcast_to(r & (L - 1), (L,))
                rowv = jnp.take_along_axis(gidm, lane, axis=0)
                for g in range(F // L):
                    s = pl.ds(g * L, L)
                    cval = plsc.load_gather(cond_v, [rowv, cols[g]])
                    out_v[b, r, s] = hits_v[b, r, s] - cval

            stores.append(pltpu.async_copy(
                out_v.at[b], out_hbm.at[pl.ds(row0, CHUNK)], osem))
            if j + NBUF < N_CHUNKS:
                start_hits(j + NBUF)

        for j in range(N_CHUNKS - NBUF, N_CHUNKS):
            stores[j].wait()

    @pl.when(hi - start >= CAP)
    def _general():
        for j in range(N_CHUNKS):
            row0 = base + j * CHUNK
            gather = pltpu.async_copy(cond_hbm.at[idx_v.at[j]],
                                      cond_v.at[pl.ds(0, CHUNK)], gsem)
            pltpu.sync_copy(hits_hbm.at[pl.ds(row0, CHUNK)], hits_v.at[0])
            gather.wait()

            @plsc.parallel_loop(0, CHUNK, 1)
            def sub_row(r):
                for g in range(F // L):
                    s = pl.ds(g * L, L)
                    out_v[0, r, s] = hits_v[0, r, s] - cond_v[r, s]

            pltpu.sync_copy(out_v.at[0], out_hbm.at[pl.ds(row0, CHUNK)])


@jax.jit
def _call(hits, cond, gid2d):
    mesh = plsc.VectorSubcoreMesh(core_axis_name="c", subcore_axis_name="s")
    k = pl.kernel(
        _sc_body,
        mesh=mesh,
        compiler_params=pltpu.CompilerParams(needs_layout_passes=False),
        out_type=jax.ShapeDtypeStruct((TOTAL, F), jnp.float32),
        scratch_types=[
            pltpu.VMEM((N_CHUNKS, CHUNK), jnp.int32),
            pltpu.VMEM((CAP, F), jnp.float32),
            pltpu.VMEM((NBUF, CHUNK, F), jnp.float32),
            pltpu.VMEM((NBUF, CHUNK, F), jnp.float32),
            pltpu.SemaphoreType.DMA,
            pltpu.SemaphoreType.DMA,
            pltpu.SemaphoreType.DMA,
            pltpu.SemaphoreType.DMA,
        ],
    )
    return k(hits, cond, gid2d)


def kernel(hits, cond, group_ids):
    gid2d = group_ids.astype(jnp.int32).reshape(TOTAL // CHUNK, CHUNK)
    return _call(hits, cond, gid2d)
